# baseline (speedup 1.0000x reference)
import jax
import jax.numpy as jnp
from jax.experimental import pallas as pl
from jax.experimental.pallas import tpu as pltpu


_TOP_K = 8
_BLOCK_T = 1024


def _router_block(x_ref, m_ref, w_ref, logits_ref, probs_ref, wts_ref, idx_ref):
    x = x_ref[...]
    w = w_ref[...]
    m = m_ref[...]
    e = w.shape[0]
    bt = x.shape[0]

    raw = jax.lax.dot_general(
        w, x, (((1,), (1,)), ((), ())), preferred_element_type=jnp.float32
    )
    logits = raw * (m * m)
    logits_ref[...] = logits

    mx = jnp.max(logits, axis=0, keepdims=True)
    ex = jnp.exp(logits - mx)
    sm = ex / jnp.sum(ex, axis=0, keepdims=True)
    probs_ref[...] = sm * m

    iota = jax.lax.broadcasted_iota(jnp.int32, (e, bt), 0)
    cur = sm
    vals = []
    idxs = []
    for _ in range(_TOP_K):
        v = jnp.max(cur, axis=0, keepdims=True)
        cand = jnp.where(cur == v, iota, e)
        ix = jnp.min(cand, axis=0, keepdims=True)
        vals.append(v)
        idxs.append(ix)
        cur = jnp.where(iota == ix, jnp.float32(-1.0), cur)
    wv = jnp.concatenate(vals, axis=0)
    iv = jnp.concatenate(idxs, axis=0)

    s = jnp.sum(wv, axis=0, keepdims=True)
    wv = wv / jnp.where(s > 0, s, jnp.ones_like(s))
    wts_ref[...] = wv * m
    idx_ref[...] = jnp.where(m != 0.0, iv, -1)


def kernel(x, x_mask, W):
    b, t, c = x.shape
    e = W.shape[0]
    n = b * t
    x2 = x.reshape(n, c)
    m2 = x_mask.reshape(1, n)

    grid = (n // _BLOCK_T,)
    logits_t, probs_t, wts_t, idx_t = pl.pallas_call(
        _router_block,
        grid=grid,
        in_specs=[
            pl.BlockSpec((_BLOCK_T, c), lambda i: (i, 0)),
            pl.BlockSpec((1, _BLOCK_T), lambda i: (0, i)),
            pl.BlockSpec((e, c), lambda i: (0, 0)),
        ],
        out_specs=[
            pl.BlockSpec((e, _BLOCK_T), lambda i: (0, i)),
            pl.BlockSpec((e, _BLOCK_T), lambda i: (0, i)),
            pl.BlockSpec((_TOP_K, _BLOCK_T), lambda i: (0, i)),
            pl.BlockSpec((_TOP_K, _BLOCK_T), lambda i: (0, i)),
        ],
        out_shape=[
            jax.ShapeDtypeStruct((e, n), jnp.float32),
            jax.ShapeDtypeStruct((e, n), jnp.float32),
            jax.ShapeDtypeStruct((_TOP_K, n), jnp.float32),
            jax.ShapeDtypeStruct((_TOP_K, n), jnp.int32),
        ],
        compiler_params=pltpu.CompilerParams(
            dimension_semantics=("arbitrary",),
        ),
    )(x2, m2, W)

    return (
        wts_t.T.reshape(b, t, _TOP_K),
        idx_t.T.reshape(b, t, _TOP_K),
        logits_t.T.reshape(b, t, e),
        probs_t.T.reshape(b, t, e),
    )
